# Initial kernel scaffold; baseline (speedup 1.0000x reference)
import jax
import jax.numpy as jnp
from jax.experimental import pallas as pl


def kernel(user_feats, graph_node_features, graph_edge_index, indices, ue_W1, ue_b1, ue_W2, ue_b2, emb_table, W_ih0, W_hh0, b_ih0, b_hh0, W_ih1, W_hh1, b_ih1, b_hh1, h0, conv1_W, conv1_b, mean_W, mean_b, logvar_W, logvar_b, eps):
    raise NotImplementedError("write your pallas kernel here")


# speedup vs baseline: 4.1703x; 4.1703x over previous
import functools

import jax
import jax.numpy as jnp
from jax import lax
from jax.experimental import pallas as pl
from jax.experimental.pallas import tpu as pltpu
from jax.experimental.pallas import tpu_sc as plsc

N_TWEETS = 3000
N_USERS = 3000
N_NODES = 6000
N_EDGES = 192000
SEQ_LEN = 30
EMBED_DIM = 128
TWEET_OUT = 100
H1 = 64
H2 = 100
BS = 1024

NPAD = 6144
TPAD = 3072
TBLK = 256
NBLK = 512
N2 = float(N_NODES) * float(N_NODES)

NC, NS = 2, 16
NW = NC * NS
CHUNK = 120

EG_CHUNKS = 24
ED_CHUNKS = 50
E_ENT = 199680
EN_CHUNKS = 52
N_ENT = N_EDGES + N_NODES
POS_WEIGHT = (N2 - N_ENT) / N_ENT

@functools.cache
def _mesh():
    return plsc.VectorSubcoreMesh(core_axis_name="c", subcore_axis_name="s",
                                  num_cores=NC)


def _wid():
    return lax.axis_index("s") * NC + lax.axis_index("c")


def _sc_gather(table, idx_groups, out_rows, nchunks):
    d = table.shape[1]

    @functools.partial(
        pl.kernel,
        mesh=_mesh(),
        out_type=jax.ShapeDtypeStruct((out_rows, d), table.dtype),
        scratch_types=[
            pltpu.VMEM((nchunks, CHUNK), jnp.int32),
            pltpu.VMEM((CHUNK, d), table.dtype),
            pltpu.SemaphoreType.DMA,
        ],
    )
    def k(tab_hbm, idx_hbm, out_hbm, idx_v, rows_v, sem):
        w = _wid()
        pltpu.sync_copy(idx_hbm.at[w], idx_v)
        base = w * (nchunks * CHUNK)

        @pl.loop(0, nchunks)
        def _(j):
            pltpu.async_copy(tab_hbm.at[idx_v.at[j]], rows_v, sem).wait()
            pltpu.sync_copy(rows_v, out_hbm.at[pl.ds(base + j * CHUNK, CHUNK)])

    return k(table, idx_groups)


def _sc_degree(dst_groups, zeros, ones):
    @functools.partial(
        pl.kernel,
        mesh=_mesh(),
        out_type=jax.ShapeDtypeStruct((NC, NPAD, 128), jnp.float32),
        scratch_types=[
            pltpu.VMEM((ED_CHUNKS, CHUNK), jnp.int32),
            pltpu.VMEM((CHUNK, 128), jnp.float32),
            pltpu.VMEM_SHARED((NPAD, 128), jnp.float32),
        ],
    )
    def k(dst_hbm, z_hbm, ones_hbm, out_hbm, idx_v, ones_v, acc):
        c = lax.axis_index("c")
        s = lax.axis_index("s")
        w = s * NC + c
        pltpu.sync_copy(dst_hbm.at[w], idx_v)
        pltpu.sync_copy(ones_hbm, ones_v)

        @pl.when(s == 0)
        def _():
            pltpu.sync_copy(z_hbm, acc)

        plsc.subcore_barrier()

        @pl.loop(0, ED_CHUNKS)
        def _(j):
            pltpu.sync_copy(ones_v, acc.at[idx_v.at[j]], add=True)

        plsc.subcore_barrier()
        rows = NPAD // NS
        pltpu.sync_copy(acc.at[pl.ds(s * rows, rows)],
                        out_hbm.at[c, pl.ds(s * rows, rows)])

    return k(dst_groups, zeros, ones)


def _sc_edge_agg(table, src_groups, dst_groups, zeros):
    d = table.shape[1]

    @functools.partial(
        pl.kernel,
        mesh=_mesh(),
        out_type=jax.ShapeDtypeStruct((NC, NPAD, d), jnp.float32),
        scratch_types=[
            pltpu.VMEM((ED_CHUNKS, CHUNK), jnp.int32),
            pltpu.VMEM((ED_CHUNKS, CHUNK), jnp.int32),
            pltpu.VMEM((CHUNK, d), jnp.float32),
            pltpu.VMEM_SHARED((NPAD, d), jnp.float32),
            pltpu.SemaphoreType.DMA,
        ],
    )
    def k(tab_hbm, src_hbm, dst_hbm, z_hbm, out_hbm,
          src_v, dst_v, rows_v, acc, sem):
        c = lax.axis_index("c")
        s = lax.axis_index("s")
        w = s * NC + c
        pltpu.sync_copy(src_hbm.at[w], src_v)
        pltpu.sync_copy(dst_hbm.at[w], dst_v)

        @pl.when(s == 0)
        def _():
            pltpu.sync_copy(z_hbm, acc)

        plsc.subcore_barrier()

        @pl.loop(0, ED_CHUNKS)
        def _(j):
            pltpu.async_copy(tab_hbm.at[src_v.at[j]], rows_v, sem).wait()
            pltpu.sync_copy(rows_v, acc.at[dst_v.at[j]], add=True)

        plsc.subcore_barrier()
        rows = NPAD // NS
        pltpu.sync_copy(acc.at[pl.ds(s * rows, rows)],
                        out_hbm.at[c, pl.ds(s * rows, rows)])

    return k(table, src_groups, dst_groups, zeros)


def _tc_user_encoder(uf, w1t, b1, w2t, b2):
    def body(uf_ref, w1_ref, b1_ref, w2_ref, b2_ref, o_ref):
        uh = jnp.maximum(
            jnp.dot(uf_ref[...], w1_ref[...],
                    preferred_element_type=jnp.float32) + b1_ref[...], 0.0)
        o_ref[...] = jnp.dot(uh, w2_ref[...],
                             preferred_element_type=jnp.float32) + b2_ref[...]

    return pl.pallas_call(
        body,
        out_shape=jax.ShapeDtypeStruct((N_USERS, 100), jnp.float32),
    )(uf, w1t, b1, w2t, b2)


def _tc_gru(emb, h0p, wih0t, whh0t, bih0, bhh0, wih1t, whh1t, bih1, bhh1):
    g3 = 3 * TWEET_OUT

    def gates(gi, gh, h):
        r = jax.nn.sigmoid(gi[:, 0:100] + gh[:, 0:100])
        z = jax.nn.sigmoid(gi[:, 100:200] + gh[:, 100:200])
        ng = jnp.tanh(gi[:, 200:300] + r * gh[:, 200:300])
        return (1.0 - z) * ng + z * h

    def body(emb_ref, h0_ref, wih0_ref, whh0_ref, bih0_ref, bhh0_ref,
             wih1_ref, whh1_ref, bih1_ref, bhh1_ref, o_ref, gi_s, out0_s):
        x = emb_ref[...].reshape(SEQ_LEN * TBLK, EMBED_DIM)
        gi0 = jnp.dot(x, wih0_ref[...],
                      preferred_element_type=jnp.float32) + bih0_ref[...]
        gi_s[...] = gi0.reshape(SEQ_LEN, TBLK, g3)

        def step0(t, h):
            gh = jnp.dot(h, whh0_ref[...],
                         preferred_element_type=jnp.float32) + bhh0_ref[...]
            hn = gates(gi_s[t], gh, h)
            out0_s[t] = hn
            return hn

        lax.fori_loop(0, SEQ_LEN, step0, h0_ref[0])

        o0 = out0_s[...].reshape(SEQ_LEN * TBLK, TWEET_OUT)
        gi1 = jnp.dot(o0, wih1_ref[...],
                      preferred_element_type=jnp.float32) + bih1_ref[...]
        gi_s[...] = gi1.reshape(SEQ_LEN, TBLK, g3)

        def step1(t, h):
            gh = jnp.dot(h, whh1_ref[...],
                         preferred_element_type=jnp.float32) + bhh1_ref[...]
            return gates(gi_s[t], gh, h)

        o_ref[...] = lax.fori_loop(0, SEQ_LEN, step1, h0_ref[1])

    grid = TPAD // TBLK
    return pl.pallas_call(
        body,
        grid=(grid,),
        in_specs=[
            pl.BlockSpec((SEQ_LEN, TBLK, EMBED_DIM), lambda i: (0, i, 0)),
            pl.BlockSpec((2, TBLK, TWEET_OUT), lambda i: (0, i, 0)),
            pl.BlockSpec((EMBED_DIM, g3), lambda i: (0, 0)),
            pl.BlockSpec((TWEET_OUT, g3), lambda i: (0, 0)),
            pl.BlockSpec((1, g3), lambda i: (0, 0)),
            pl.BlockSpec((1, g3), lambda i: (0, 0)),
            pl.BlockSpec((TWEET_OUT, g3), lambda i: (0, 0)),
            pl.BlockSpec((TWEET_OUT, g3), lambda i: (0, 0)),
            pl.BlockSpec((1, g3), lambda i: (0, 0)),
            pl.BlockSpec((1, g3), lambda i: (0, 0)),
        ],
        out_specs=pl.BlockSpec((TBLK, TWEET_OUT), lambda i: (i, 0)),
        out_shape=jax.ShapeDtypeStruct((TPAD, TWEET_OUT), jnp.float32),
        scratch_shapes=[
            pltpu.VMEM((SEQ_LEN, TBLK, g3), jnp.float32),
            pltpu.VMEM((SEQ_LEN, TBLK, TWEET_OUT), jnp.float32),
        ],
    )(emb, h0p, wih0t, whh0t, bih0, bhh0, wih1t, whh1t, bih1, bhh1)


def _tc_stage_a(x_in, w1t, degp):
    def body(x_ref, w_ref, deg_ref, xw_ref, dinv_ref):
        deg = deg_ref[0, :, 0:1] + deg_ref[1, :, 0:1] + 1.0
        dinv = lax.rsqrt(deg)
        dinv_ref[...] = dinv
        xw = jnp.dot(x_ref[...], w_ref[...], preferred_element_type=jnp.float32)
        xw_ref[...] = dinv * xw

    return pl.pallas_call(
        body,
        out_shape=(
            jax.ShapeDtypeStruct((NPAD, 128), jnp.float32),
            jax.ShapeDtypeStruct((NPAD, 1), jnp.float32),
        ),
    )(x_in, w1t, degp)


def _tc_stage_b(agg1, xw1s, dinv, b1, w2mt, w2lt):
    def body(agg_ref, xw_ref, dinv_ref, b1_ref, wm_ref, wl_ref, om_ref, ol_ref):
        dinv = dinv_ref[...]
        pre = dinv * (agg_ref[0] + agg_ref[1] + xw_ref[...]) + b1_ref[...]
        x = jnp.where(pre > 0.0, pre, jnp.exp(jnp.minimum(pre, 0.0)) - 1.0)
        x = x[:, 0:H1]
        om_ref[...] = dinv * jnp.dot(x, wm_ref[...],
                                     preferred_element_type=jnp.float32)
        ol_ref[...] = dinv * jnp.dot(x, wl_ref[...],
                                     preferred_element_type=jnp.float32)

    return pl.pallas_call(
        body,
        out_shape=(
            jax.ShapeDtypeStruct((NPAD, 128), jnp.float32),
            jax.ShapeDtypeStruct((NPAD, 128), jnp.float32),
        ),
    )(agg1, xw1s, dinv, b1, w2mt, w2lt)


def _tc_stage_c(agg_m, agg_l, xwm, xwl, dinv, bm, bl, epsp):
    def body(aggm_ref, aggl_ref, xwm_ref, xwl_ref, dinv_ref, bm_ref, bl_ref,
             eps_ref, z_ref, zbf_ref, kl_ref):
        dinv = dinv_ref[...]
        pre_m = dinv * (aggm_ref[0] + aggm_ref[1] + xwm_ref[...]) + bm_ref[...]
        pre_l = dinv * (aggl_ref[0] + aggl_ref[1] + xwl_ref[...]) + bl_ref[...]
        mu = jnp.maximum(pre_m[:, 0:100], 0.0)
        logvar = jnp.maximum(pre_l[:, 0:100], 0.0)
        rows = lax.broadcasted_iota(jnp.int32, (NPAD, 1), 0)
        rmask = rows < N_NODES
        z = mu + jnp.exp(logvar * 0.5) * eps_ref[...]
        z = jnp.where(rmask, z, 0.0)
        z_ref[...] = z
        zbf_ref[...] = jnp.concatenate(
            [z, jnp.zeros((NPAD, 28), jnp.float32)], axis=1)
        klds = -0.5 * (1.0 + logvar - mu * mu - jnp.exp(logvar))
        klds = jnp.where(rmask, klds, 0.0)
        kl_ref[0, 0] = jnp.sum(klds)

    return pl.pallas_call(
        body,
        out_shape=(
            jax.ShapeDtypeStruct((NPAD, 100), jnp.float32),
            jax.ShapeDtypeStruct((NPAD, 128), jnp.float32),
            jax.ShapeDtypeStruct((1, 1), jnp.float32),
        ),
        out_specs=(
            pl.BlockSpec(memory_space=pltpu.VMEM),
            pl.BlockSpec(memory_space=pltpu.VMEM),
            pl.BlockSpec(memory_space=pltpu.SMEM),
        ),
    )(agg_m, agg_l, xwm, xwl, dinv, bm, bl, epsp)


def _tc_decoder(zbf):
    g = NPAD // NBLK

    def body(zi_ref, zj_ref, s1_ref, s2_ref):
        i = pl.program_id(0)
        j = pl.program_id(1)

        @pl.when(jnp.logical_and(i == 0, j == 0))
        def _():
            s1_ref[0, 0] = 0.0
            s2_ref[0, 0] = 0.0

        p = lax.dot_general(zi_ref[...].astype(jnp.bfloat16),
                            zj_ref[...].astype(jnp.bfloat16),
                            (((1,), (1,)), ((), ())),
                            preferred_element_type=jnp.float32)
        rows = i * NBLK + lax.broadcasted_iota(jnp.int32, (NBLK, NBLK), 0)
        cols = j * NBLK + lax.broadcasted_iota(jnp.int32, (NBLK, NBLK), 1)
        valid = jnp.logical_and(rows < N_NODES, cols < N_NODES)
        sig = jax.nn.sigmoid(p)
        s1_ref[0, 0] += jnp.sum(jnp.where(valid, sig, 0.0))
        pr = jnp.clip(sig, 1e-7, 1.0 - 1e-7)
        t2 = -jnp.log(1.0 - pr)
        s2_ref[0, 0] += jnp.sum(jnp.where(valid, t2, 0.0))

    return pl.pallas_call(
        body,
        grid=(g, g),
        in_specs=[
            pl.BlockSpec((NBLK, 128), lambda i, j: (i, 0)),
            pl.BlockSpec((NBLK, 128), lambda i, j: (j, 0)),
        ],
        out_specs=(
            pl.BlockSpec(memory_space=pltpu.SMEM),
            pl.BlockSpec(memory_space=pltpu.SMEM),
        ),
        out_shape=(
            jax.ShapeDtypeStruct((1, 1), jnp.float32),
            jax.ShapeDtypeStruct((1, 1), jnp.float32),
        ),
    )(zbf, zbf)


def _tc_corrections(za, zb, keys3):
    eblk = 512
    g = E_ENT // eblk

    def body(za_ref, zb_ref, k_ref, o_ref):
        i = pl.program_id(0)

        @pl.when(i == 0)
        def _():
            o_ref[0, 0] = 0.0

        prod = za_ref[...] * zb_ref[...]
        d = jnp.sum(prod, axis=1, keepdims=True)
        k0 = k_ref[0]
        singleton = jnp.logical_and(k0 != k_ref[1], k0 != k_ref[2])
        pos = i * eblk + lax.broadcasted_iota(jnp.int32, (eblk, 1), 0)
        valid = pos < N_ENT
        p = jnp.clip(jax.nn.sigmoid(d), 1e-7, 1.0 - 1e-7)
        logp = jnp.log(p)
        log1mp = jnp.log(1.0 - p)
        b0 = -log1mp
        ell = log1mp - logp
        contrib = jnp.where(singleton,
                            POS_WEIGHT * (b0 + ell) - b0, ell)
        o_ref[0, 0] += jnp.sum(jnp.where(valid, contrib, 0.0))

    return pl.pallas_call(
        body,
        grid=(g,),
        in_specs=[
            pl.BlockSpec((eblk, 128), lambda i: (i, 0)),
            pl.BlockSpec((eblk, 128), lambda i: (i, 0)),
            pl.BlockSpec((3, eblk, 1), lambda i: (0, i, 0)),
        ],
        out_specs=pl.BlockSpec(memory_space=pltpu.SMEM),
        out_shape=jax.ShapeDtypeStruct((1, 1), jnp.float32),
    )(za, zb, keys3)


def kernel(user_feats, graph_node_features, graph_edge_index, indices,
           ue_W1, ue_b1, ue_W2, ue_b2, emb_table,
           W_ih0, W_hh0, b_ih0, b_hh0, W_ih1, W_hh1, b_ih1, b_hh1, h0,
           conv1_W, conv1_b, mean_W, mean_b, logvar_W, logvar_b, eps):
    bs = indices.shape[0]
    f32 = jnp.float32

    tok = graph_node_features.astype(jnp.int32).T
    tok = jnp.pad(tok, ((0, 0), (0, TPAD - N_TWEETS)))
    tok_g = tok.reshape(NW, EG_CHUNKS, CHUNK)
    emb = _sc_gather(emb_table, tok_g, SEQ_LEN * TPAD, EG_CHUNKS)
    emb = emb.reshape(SEQ_LEN, TPAD, EMBED_DIM)

    user_emb = _tc_user_encoder(
        user_feats, ue_W1.T, ue_b1.reshape(1, -1), ue_W2.T, ue_b2.reshape(1, -1))

    h0p = jnp.pad(h0, ((0, 0), (0, TPAD - N_TWEETS), (0, 0)))
    hn = _tc_gru(emb, h0p,
                 W_ih0.T, W_hh0.T, b_ih0.reshape(1, -1), b_hh0.reshape(1, -1),
                 W_ih1.T, W_hh1.T, b_ih1.reshape(1, -1), b_hh1.reshape(1, -1))

    src = graph_edge_index[0].astype(jnp.int32)
    dst = graph_edge_index[1].astype(jnp.int32)
    src_g = src.reshape(NW, ED_CHUNKS, CHUNK)
    dst_g = dst.reshape(NW, ED_CHUNKS, CHUNK)

    ones128 = jnp.ones((CHUNK, 128), f32)
    degp = _sc_degree(dst_g, jnp.zeros((NPAD, 128), f32), ones128)

    x_in = jnp.concatenate([hn[:bs], user_emb, hn[bs:N_TWEETS]], axis=0)
    x_in = jnp.pad(x_in, ((0, NPAD - N_NODES), (0, 0)))
    w1t = jnp.pad(conv1_W.T, ((0, 0), (0, 128 - H1)))
    xw1s, dinv = _tc_stage_a(x_in, w1t, degp)
    agg1 = _sc_edge_agg(xw1s, src_g, dst_g, jnp.zeros((NPAD, 128), f32))

    w2mt = jnp.pad(mean_W.T, ((0, 0), (0, 28)))
    w2lt = jnp.pad(logvar_W.T, ((0, 0), (0, 28)))
    b1p = jnp.pad(conv1_b.reshape(1, -1), ((0, 0), (0, 128 - H1)))
    xwm, xwl = _tc_stage_b(agg1, xw1s, dinv, b1p, w2mt, w2lt)
    zeros128 = jnp.zeros((NPAD, 128), f32)
    agg_m = _sc_edge_agg(xwm, src_g, dst_g, zeros128)
    agg_l = _sc_edge_agg(xwl, src_g, dst_g, zeros128)

    bm = jnp.pad(mean_b.reshape(1, -1), ((0, 0), (0, 28)))
    bl = jnp.pad(logvar_b.reshape(1, -1), ((0, 0), (0, 28)))
    epsp = jnp.pad(eps, ((0, NPAD - N_NODES), (0, 0)))
    zf, zbf, klsum = _tc_stage_c(agg_m, agg_l, xwm, xwl, dinv, bm, bl, epsp)
    kl_loss = (klsum / float(N_NODES)).reshape(1)

    diag = jnp.arange(N_NODES, dtype=jnp.int32) * (N_NODES + 1)
    keys = jnp.concatenate([
        src * N_NODES + dst, diag,
        jnp.full((E_ENT - N_ENT,), jnp.iinfo(jnp.int32).max, jnp.int32)])
    keys = jnp.sort(keys)
    kc = jnp.minimum(keys, N_NODES * N_NODES - 1)
    a_idx = (kc // N_NODES).reshape(NW, EN_CHUNKS, CHUNK)
    b_idx = (kc % N_NODES).reshape(NW, EN_CHUNKS, CHUNK)
    kprev = jnp.concatenate([jnp.full((1,), -1, jnp.int32), keys[:-1]])
    knext = jnp.concatenate([keys[1:], jnp.full((1,), -2, jnp.int32)])
    keys3 = jnp.stack([keys, kprev, knext]).reshape(3, E_ENT, 1)

    za = _sc_gather(zbf, a_idx, E_ENT, EN_CHUNKS)
    zb = _sc_gather(zbf, b_idx, E_ENT, EN_CHUNKS)

    s1, s2 = _tc_decoder(zbf)
    corr = _tc_corrections(za, zb, keys3)

    s1v = s1[0, 0]
    norm = N2 / ((N2 - s1v) * 2.0)
    rec_loss = norm * (s2[0, 0] + corr[0, 0]) / N2

    return (zf[:bs, :], kl_loss, rec_loss)


# speedup vs baseline: 4.3238x; 1.0368x over previous
import functools

import jax
import jax.numpy as jnp
from jax import lax
from jax.experimental import pallas as pl
from jax.experimental.pallas import tpu as pltpu
from jax.experimental.pallas import tpu_sc as plsc

N_TWEETS = 3000
N_USERS = 3000
N_NODES = 6000
N_EDGES = 192000
SEQ_LEN = 30
EMBED_DIM = 128
TWEET_OUT = 100
H1 = 64
H2 = 100
BS = 1024

NPAD = 6144
TPAD = 3072
TBLK = 256
NBLK = 512
N2 = float(N_NODES) * float(N_NODES)

NC, NS = 2, 16
NW = NC * NS
CHUNK = 120

EG_CHUNKS = 24
ED_CHUNKS = 50
E_ENT = 199680
EN_CHUNKS = 52
N_ENT = N_EDGES + N_NODES
POS_WEIGHT = (N2 - N_ENT) / N_ENT

@functools.cache
def _mesh():
    return plsc.VectorSubcoreMesh(core_axis_name="c", subcore_axis_name="s",
                                  num_cores=NC)


def _wid():
    return lax.axis_index("s") * NC + lax.axis_index("c")


def _sc_gather(table, idx_groups, out_rows, nchunks):
    d = table.shape[1]

    assert nchunks % 2 == 0

    @functools.partial(
        pl.kernel,
        mesh=_mesh(),
        out_type=jax.ShapeDtypeStruct((out_rows, d), table.dtype),
        scratch_types=[
            pltpu.VMEM((nchunks, CHUNK), jnp.int32),
            pltpu.VMEM((CHUNK, d), table.dtype),
            pltpu.VMEM((CHUNK, d), table.dtype),
            pltpu.SemaphoreType.DMA,
            pltpu.SemaphoreType.DMA,
        ],
    )
    def k(tab_hbm, idx_hbm, out_hbm, idx_v, rows_a, rows_b, sem_a, sem_b):
        w = _wid()
        pltpu.sync_copy(idx_hbm.at[w], idx_v)
        base = w * (nchunks * CHUNK)

        pltpu.async_copy(tab_hbm.at[idx_v.at[0]], rows_a, sem_a)

        @pl.loop(0, nchunks, step=2)
        def _(j):
            pltpu.make_async_copy(tab_hbm.at[idx_v.at[j]], rows_a, sem_a).wait()
            pltpu.async_copy(tab_hbm.at[idx_v.at[j + 1]], rows_b, sem_b)
            pltpu.sync_copy(rows_a, out_hbm.at[pl.ds(base + j * CHUNK, CHUNK)])
            pltpu.make_async_copy(tab_hbm.at[idx_v.at[j + 1]], rows_b,
                                  sem_b).wait()

            @pl.when(j + 2 < nchunks)
            def _():
                pltpu.async_copy(tab_hbm.at[idx_v.at[j + 2]], rows_a, sem_a)

            pltpu.sync_copy(rows_b,
                            out_hbm.at[pl.ds(base + (j + 1) * CHUNK, CHUNK)])

    return k(table, idx_groups)


def _sc_degree(dst_groups, zeros, ones):
    @functools.partial(
        pl.kernel,
        mesh=_mesh(),
        out_type=jax.ShapeDtypeStruct((NC, NPAD, 128), jnp.float32),
        scratch_types=[
            pltpu.VMEM((ED_CHUNKS, CHUNK), jnp.int32),
            pltpu.VMEM((CHUNK, 128), jnp.float32),
            pltpu.VMEM_SHARED((NPAD, 128), jnp.float32),
        ],
    )
    def k(dst_hbm, z_hbm, ones_hbm, out_hbm, idx_v, ones_v, acc):
        c = lax.axis_index("c")
        s = lax.axis_index("s")
        w = s * NC + c
        pltpu.sync_copy(dst_hbm.at[w], idx_v)
        pltpu.sync_copy(ones_hbm, ones_v)

        @pl.when(s == 0)
        def _():
            pltpu.sync_copy(z_hbm, acc)

        plsc.subcore_barrier()

        @pl.loop(0, ED_CHUNKS)
        def _(j):
            pltpu.sync_copy(ones_v, acc.at[idx_v.at[j]], add=True)

        plsc.subcore_barrier()
        rows = NPAD // NS
        pltpu.sync_copy(acc.at[pl.ds(s * rows, rows)],
                        out_hbm.at[c, pl.ds(s * rows, rows)])

    return k(dst_groups, zeros, ones)


def _sc_edge_agg(table, src_groups, dst_groups, zeros):
    d = table.shape[1]

    @functools.partial(
        pl.kernel,
        mesh=_mesh(),
        out_type=jax.ShapeDtypeStruct((NC, NPAD, d), jnp.float32),
        scratch_types=[
            pltpu.VMEM((ED_CHUNKS, CHUNK), jnp.int32),
            pltpu.VMEM((ED_CHUNKS, CHUNK), jnp.int32),
            pltpu.VMEM((CHUNK, d), jnp.float32),
            pltpu.VMEM((CHUNK, d), jnp.float32),
            pltpu.VMEM_SHARED((NPAD, d), jnp.float32),
            pltpu.SemaphoreType.DMA,
            pltpu.SemaphoreType.DMA,
        ],
    )
    def k(tab_hbm, src_hbm, dst_hbm, z_hbm, out_hbm,
          src_v, dst_v, rows_a, rows_b, acc, sem_a, sem_b):
        c = lax.axis_index("c")
        s = lax.axis_index("s")
        w = s * NC + c
        pltpu.sync_copy(src_hbm.at[w], src_v)
        pltpu.sync_copy(dst_hbm.at[w], dst_v)

        @pl.when(s == 0)
        def _():
            pltpu.sync_copy(z_hbm, acc)

        plsc.subcore_barrier()

        pltpu.async_copy(tab_hbm.at[src_v.at[0]], rows_a, sem_a)

        @pl.loop(0, ED_CHUNKS, step=2)
        def _(j):
            pltpu.make_async_copy(tab_hbm.at[src_v.at[j]], rows_a, sem_a).wait()
            pltpu.async_copy(tab_hbm.at[src_v.at[j + 1]], rows_b, sem_b)
            pltpu.sync_copy(rows_a, acc.at[dst_v.at[j]], add=True)
            pltpu.make_async_copy(tab_hbm.at[src_v.at[j + 1]], rows_b,
                                  sem_b).wait()

            @pl.when(j + 2 < ED_CHUNKS)
            def _():
                pltpu.async_copy(tab_hbm.at[src_v.at[j + 2]], rows_a, sem_a)

            pltpu.sync_copy(rows_b, acc.at[dst_v.at[j + 1]], add=True)

        plsc.subcore_barrier()
        rows = NPAD // NS
        pltpu.sync_copy(acc.at[pl.ds(s * rows, rows)],
                        out_hbm.at[c, pl.ds(s * rows, rows)])

    return k(table, src_groups, dst_groups, zeros)


def _tc_user_encoder(uf, w1t, b1, w2t, b2):
    def body(uf_ref, w1_ref, b1_ref, w2_ref, b2_ref, o_ref):
        uh = jnp.maximum(
            jnp.dot(uf_ref[...], w1_ref[...],
                    preferred_element_type=jnp.float32) + b1_ref[...], 0.0)
        o_ref[...] = jnp.dot(uh, w2_ref[...],
                             preferred_element_type=jnp.float32) + b2_ref[...]

    return pl.pallas_call(
        body,
        out_shape=jax.ShapeDtypeStruct((N_USERS, 100), jnp.float32),
    )(uf, w1t, b1, w2t, b2)


def _tc_gru(emb, h0p, wih0t, whh0t, bih0, bhh0, wih1t, whh1t, bih1, bhh1):
    g3 = 3 * TWEET_OUT

    def gates(gi, gh, h):
        r = jax.nn.sigmoid(gi[:, 0:100] + gh[:, 0:100])
        z = jax.nn.sigmoid(gi[:, 100:200] + gh[:, 100:200])
        ng = jnp.tanh(gi[:, 200:300] + r * gh[:, 200:300])
        return (1.0 - z) * ng + z * h

    def body(emb_ref, h0_ref, wih0_ref, whh0_ref, bih0_ref, bhh0_ref,
             wih1_ref, whh1_ref, bih1_ref, bhh1_ref, o_ref, gi_s, out0_s):
        x = emb_ref[...].reshape(SEQ_LEN * TBLK, EMBED_DIM)
        gi0 = jnp.dot(x, wih0_ref[...],
                      preferred_element_type=jnp.float32) + bih0_ref[...]
        gi_s[...] = gi0.reshape(SEQ_LEN, TBLK, g3)

        def step0(t, h):
            gh = jnp.dot(h, whh0_ref[...],
                         preferred_element_type=jnp.float32) + bhh0_ref[...]
            hn = gates(gi_s[t], gh, h)
            out0_s[t] = hn
            return hn

        lax.fori_loop(0, SEQ_LEN, step0, h0_ref[0])

        o0 = out0_s[...].reshape(SEQ_LEN * TBLK, TWEET_OUT)
        gi1 = jnp.dot(o0, wih1_ref[...],
                      preferred_element_type=jnp.float32) + bih1_ref[...]
        gi_s[...] = gi1.reshape(SEQ_LEN, TBLK, g3)

        def step1(t, h):
            gh = jnp.dot(h, whh1_ref[...],
                         preferred_element_type=jnp.float32) + bhh1_ref[...]
            return gates(gi_s[t], gh, h)

        o_ref[...] = lax.fori_loop(0, SEQ_LEN, step1, h0_ref[1])

    grid = TPAD // TBLK
    return pl.pallas_call(
        body,
        grid=(grid,),
        in_specs=[
            pl.BlockSpec((SEQ_LEN, TBLK, EMBED_DIM), lambda i: (0, i, 0)),
            pl.BlockSpec((2, TBLK, TWEET_OUT), lambda i: (0, i, 0)),
            pl.BlockSpec((EMBED_DIM, g3), lambda i: (0, 0)),
            pl.BlockSpec((TWEET_OUT, g3), lambda i: (0, 0)),
            pl.BlockSpec((1, g3), lambda i: (0, 0)),
            pl.BlockSpec((1, g3), lambda i: (0, 0)),
            pl.BlockSpec((TWEET_OUT, g3), lambda i: (0, 0)),
            pl.BlockSpec((TWEET_OUT, g3), lambda i: (0, 0)),
            pl.BlockSpec((1, g3), lambda i: (0, 0)),
            pl.BlockSpec((1, g3), lambda i: (0, 0)),
        ],
        out_specs=pl.BlockSpec((TBLK, TWEET_OUT), lambda i: (i, 0)),
        out_shape=jax.ShapeDtypeStruct((TPAD, TWEET_OUT), jnp.float32),
        scratch_shapes=[
            pltpu.VMEM((SEQ_LEN, TBLK, g3), jnp.float32),
            pltpu.VMEM((SEQ_LEN, TBLK, TWEET_OUT), jnp.float32),
        ],
    )(emb, h0p, wih0t, whh0t, bih0, bhh0, wih1t, whh1t, bih1, bhh1)


def _tc_stage_a(x_in, w1t, degp):
    def body(x_ref, w_ref, deg_ref, xw_ref, dinv_ref):
        deg = deg_ref[0, :, 0:1] + deg_ref[1, :, 0:1] + 1.0
        dinv = lax.rsqrt(deg)
        dinv_ref[...] = dinv
        xw = jnp.dot(x_ref[...], w_ref[...], preferred_element_type=jnp.float32)
        xw_ref[...] = dinv * xw

    return pl.pallas_call(
        body,
        out_shape=(
            jax.ShapeDtypeStruct((NPAD, 128), jnp.float32),
            jax.ShapeDtypeStruct((NPAD, 1), jnp.float32),
        ),
    )(x_in, w1t, degp)


def _tc_stage_b(agg1, xw1s, dinv, b1, w2mt, w2lt):
    def body(agg_ref, xw_ref, dinv_ref, b1_ref, wm_ref, wl_ref, om_ref, ol_ref):
        dinv = dinv_ref[...]
        pre = dinv * (agg_ref[0] + agg_ref[1] + xw_ref[...]) + b1_ref[...]
        x = jnp.where(pre > 0.0, pre, jnp.exp(jnp.minimum(pre, 0.0)) - 1.0)
        x = x[:, 0:H1]
        om_ref[...] = dinv * jnp.dot(x, wm_ref[...],
                                     preferred_element_type=jnp.float32)
        ol_ref[...] = dinv * jnp.dot(x, wl_ref[...],
                                     preferred_element_type=jnp.float32)

    return pl.pallas_call(
        body,
        out_shape=(
            jax.ShapeDtypeStruct((NPAD, 128), jnp.float32),
            jax.ShapeDtypeStruct((NPAD, 128), jnp.float32),
        ),
    )(agg1, xw1s, dinv, b1, w2mt, w2lt)


def _tc_stage_c(agg_m, agg_l, xwm, xwl, dinv, bm, bl, epsp):
    def body(aggm_ref, aggl_ref, xwm_ref, xwl_ref, dinv_ref, bm_ref, bl_ref,
             eps_ref, z_ref, zbf_ref, kl_ref):
        dinv = dinv_ref[...]
        pre_m = dinv * (aggm_ref[0] + aggm_ref[1] + xwm_ref[...]) + bm_ref[...]
        pre_l = dinv * (aggl_ref[0] + aggl_ref[1] + xwl_ref[...]) + bl_ref[...]
        mu = jnp.maximum(pre_m[:, 0:100], 0.0)
        logvar = jnp.maximum(pre_l[:, 0:100], 0.0)
        rows = lax.broadcasted_iota(jnp.int32, (NPAD, 1), 0)
        rmask = rows < N_NODES
        z = mu + jnp.exp(logvar * 0.5) * eps_ref[...]
        z = jnp.where(rmask, z, 0.0)
        z_ref[...] = z
        zbf_ref[...] = jnp.concatenate(
            [z, jnp.zeros((NPAD, 28), jnp.float32)], axis=1)
        klds = -0.5 * (1.0 + logvar - mu * mu - jnp.exp(logvar))
        klds = jnp.where(rmask, klds, 0.0)
        kl_ref[0, 0] = jnp.sum(klds)

    return pl.pallas_call(
        body,
        out_shape=(
            jax.ShapeDtypeStruct((NPAD, 100), jnp.float32),
            jax.ShapeDtypeStruct((NPAD, 128), jnp.float32),
            jax.ShapeDtypeStruct((1, 1), jnp.float32),
        ),
        out_specs=(
            pl.BlockSpec(memory_space=pltpu.VMEM),
            pl.BlockSpec(memory_space=pltpu.VMEM),
            pl.BlockSpec(memory_space=pltpu.SMEM),
        ),
    )(agg_m, agg_l, xwm, xwl, dinv, bm, bl, epsp)


def _tc_decoder(zbf):
    g = NPAD // NBLK

    def body(zi_ref, zj_ref, s1_ref, s2_ref):
        i = pl.program_id(0)
        j = pl.program_id(1)

        @pl.when(jnp.logical_and(i == 0, j == 0))
        def _():
            s1_ref[0, 0] = 0.0
            s2_ref[0, 0] = 0.0

        p = lax.dot_general(zi_ref[...].astype(jnp.bfloat16),
                            zj_ref[...].astype(jnp.bfloat16),
                            (((1,), (1,)), ((), ())),
                            preferred_element_type=jnp.float32)
        rows = i * NBLK + lax.broadcasted_iota(jnp.int32, (NBLK, NBLK), 0)
        cols = j * NBLK + lax.broadcasted_iota(jnp.int32, (NBLK, NBLK), 1)
        valid = jnp.logical_and(rows < N_NODES, cols < N_NODES)
        sig = jax.nn.sigmoid(p)
        s1_ref[0, 0] += jnp.sum(jnp.where(valid, sig, 0.0))
        pr = jnp.clip(sig, 1e-7, 1.0 - 1e-7)
        t2 = -jnp.log(1.0 - pr)
        s2_ref[0, 0] += jnp.sum(jnp.where(valid, t2, 0.0))

    return pl.pallas_call(
        body,
        grid=(g, g),
        in_specs=[
            pl.BlockSpec((NBLK, 128), lambda i, j: (i, 0)),
            pl.BlockSpec((NBLK, 128), lambda i, j: (j, 0)),
        ],
        out_specs=(
            pl.BlockSpec(memory_space=pltpu.SMEM),
            pl.BlockSpec(memory_space=pltpu.SMEM),
        ),
        out_shape=(
            jax.ShapeDtypeStruct((1, 1), jnp.float32),
            jax.ShapeDtypeStruct((1, 1), jnp.float32),
        ),
    )(zbf, zbf)


def _tc_corrections(za, zb, keys3):
    eblk = 512
    g = E_ENT // eblk

    def body(za_ref, zb_ref, k_ref, o_ref):
        i = pl.program_id(0)

        @pl.when(i == 0)
        def _():
            o_ref[0, 0] = 0.0

        prod = za_ref[...] * zb_ref[...]
        d = jnp.sum(prod, axis=1, keepdims=True)
        k0 = k_ref[0]
        singleton = jnp.logical_and(k0 != k_ref[1], k0 != k_ref[2])
        pos = i * eblk + lax.broadcasted_iota(jnp.int32, (eblk, 1), 0)
        valid = pos < N_ENT
        p = jnp.clip(jax.nn.sigmoid(d), 1e-7, 1.0 - 1e-7)
        logp = jnp.log(p)
        log1mp = jnp.log(1.0 - p)
        b0 = -log1mp
        ell = log1mp - logp
        contrib = jnp.where(singleton,
                            POS_WEIGHT * (b0 + ell) - b0, ell)
        o_ref[0, 0] += jnp.sum(jnp.where(valid, contrib, 0.0))

    return pl.pallas_call(
        body,
        grid=(g,),
        in_specs=[
            pl.BlockSpec((eblk, 128), lambda i: (i, 0)),
            pl.BlockSpec((eblk, 128), lambda i: (i, 0)),
            pl.BlockSpec((3, eblk, 1), lambda i: (0, i, 0)),
        ],
        out_specs=pl.BlockSpec(memory_space=pltpu.SMEM),
        out_shape=jax.ShapeDtypeStruct((1, 1), jnp.float32),
    )(za, zb, keys3)


def kernel(user_feats, graph_node_features, graph_edge_index, indices,
           ue_W1, ue_b1, ue_W2, ue_b2, emb_table,
           W_ih0, W_hh0, b_ih0, b_hh0, W_ih1, W_hh1, b_ih1, b_hh1, h0,
           conv1_W, conv1_b, mean_W, mean_b, logvar_W, logvar_b, eps):
    bs = indices.shape[0]
    f32 = jnp.float32

    tok = graph_node_features.astype(jnp.int32).T
    tok = jnp.pad(tok, ((0, 0), (0, TPAD - N_TWEETS)))
    tok_g = tok.reshape(NW, EG_CHUNKS, CHUNK)
    emb = _sc_gather(emb_table, tok_g, SEQ_LEN * TPAD, EG_CHUNKS)
    emb = emb.reshape(SEQ_LEN, TPAD, EMBED_DIM)

    user_emb = _tc_user_encoder(
        user_feats, ue_W1.T, ue_b1.reshape(1, -1), ue_W2.T, ue_b2.reshape(1, -1))

    h0p = jnp.pad(h0, ((0, 0), (0, TPAD - N_TWEETS), (0, 0)))
    hn = _tc_gru(emb, h0p,
                 W_ih0.T, W_hh0.T, b_ih0.reshape(1, -1), b_hh0.reshape(1, -1),
                 W_ih1.T, W_hh1.T, b_ih1.reshape(1, -1), b_hh1.reshape(1, -1))

    src = graph_edge_index[0].astype(jnp.int32)
    dst = graph_edge_index[1].astype(jnp.int32)
    src_g = src.reshape(NW, ED_CHUNKS, CHUNK)
    dst_g = dst.reshape(NW, ED_CHUNKS, CHUNK)

    ones128 = jnp.ones((CHUNK, 128), f32)
    degp = _sc_degree(dst_g, jnp.zeros((NPAD, 128), f32), ones128)

    x_in = jnp.concatenate([hn[:bs], user_emb, hn[bs:N_TWEETS]], axis=0)
    x_in = jnp.pad(x_in, ((0, NPAD - N_NODES), (0, 0)))
    w1t = jnp.pad(conv1_W.T, ((0, 0), (0, 128 - H1)))
    xw1s, dinv = _tc_stage_a(x_in, w1t, degp)
    agg1 = _sc_edge_agg(xw1s, src_g, dst_g, jnp.zeros((NPAD, 128), f32))

    w2mt = jnp.pad(mean_W.T, ((0, 0), (0, 28)))
    w2lt = jnp.pad(logvar_W.T, ((0, 0), (0, 28)))
    b1p = jnp.pad(conv1_b.reshape(1, -1), ((0, 0), (0, 128 - H1)))
    xwm, xwl = _tc_stage_b(agg1, xw1s, dinv, b1p, w2mt, w2lt)
    zeros128 = jnp.zeros((NPAD, 128), f32)
    agg_m = _sc_edge_agg(xwm, src_g, dst_g, zeros128)
    agg_l = _sc_edge_agg(xwl, src_g, dst_g, zeros128)

    bm = jnp.pad(mean_b.reshape(1, -1), ((0, 0), (0, 28)))
    bl = jnp.pad(logvar_b.reshape(1, -1), ((0, 0), (0, 28)))
    epsp = jnp.pad(eps, ((0, NPAD - N_NODES), (0, 0)))
    zf, zbf, klsum = _tc_stage_c(agg_m, agg_l, xwm, xwl, dinv, bm, bl, epsp)
    kl_loss = (klsum / float(N_NODES)).reshape(1)

    diag = jnp.arange(N_NODES, dtype=jnp.int32) * (N_NODES + 1)
    keys = jnp.concatenate([
        src * N_NODES + dst, diag,
        jnp.full((E_ENT - N_ENT,), jnp.iinfo(jnp.int32).max, jnp.int32)])
    keys = jnp.sort(keys)
    kc = jnp.minimum(keys, N_NODES * N_NODES - 1)
    a_idx = (kc // N_NODES).reshape(NW, EN_CHUNKS, CHUNK)
    b_idx = (kc % N_NODES).reshape(NW, EN_CHUNKS, CHUNK)
    kprev = jnp.concatenate([jnp.full((1,), -1, jnp.int32), keys[:-1]])
    knext = jnp.concatenate([keys[1:], jnp.full((1,), -2, jnp.int32)])
    keys3 = jnp.stack([keys, kprev, knext]).reshape(3, E_ENT, 1)

    za = _sc_gather(zbf, a_idx, E_ENT, EN_CHUNKS)
    zb = _sc_gather(zbf, b_idx, E_ENT, EN_CHUNKS)

    s1, s2 = _tc_decoder(zbf)
    corr = _tc_corrections(za, zb, keys3)

    s1v = s1[0, 0]
    norm = N2 / ((N2 - s1v) * 2.0)
    rec_loss = norm * (s2[0, 0] + corr[0, 0]) / N2

    return (zf[:bs, :], kl_loss, rec_loss)


# speedup vs baseline: 4.5138x; 1.0440x over previous
import functools

import jax
import jax.numpy as jnp
from jax import lax
from jax.experimental import pallas as pl
from jax.experimental.pallas import tpu as pltpu
from jax.experimental.pallas import tpu_sc as plsc

N_TWEETS = 3000
N_USERS = 3000
N_NODES = 6000
N_EDGES = 192000
SEQ_LEN = 30
EMBED_DIM = 128
TWEET_OUT = 100
H1 = 64
H2 = 100
BS = 1024

NPAD = 6144
TPAD = 3072
TBLK = 512
NBLK = 512
N2 = float(N_NODES) * float(N_NODES)

NC, NS = 2, 16
NW = NC * NS
CHUNK = 120

EG_CHUNKS = 24
EG_CHUNK = 120
ED_CHUNKS = 50
E_ENT = 199680
EN_CHUNKS = 52
EN_CHUNK = 120
N_ENT = N_EDGES + N_NODES
POS_WEIGHT = (N2 - N_ENT) / N_ENT

@functools.cache
def _mesh():
    return plsc.VectorSubcoreMesh(core_axis_name="c", subcore_axis_name="s",
                                  num_cores=NC)


def _wid():
    return lax.axis_index("s") * NC + lax.axis_index("c")


def _sc_gather(table, idx_groups, out_rows, nchunks, chunk):
    d = table.shape[1]

    assert nchunks % 2 == 0

    @functools.partial(
        pl.kernel,
        mesh=_mesh(),
        out_type=jax.ShapeDtypeStruct((out_rows, d), table.dtype),
        scratch_types=[
            pltpu.VMEM((nchunks, chunk), jnp.int32),
            pltpu.VMEM((chunk, d), table.dtype),
            pltpu.VMEM((chunk, d), table.dtype),
            pltpu.SemaphoreType.DMA,
            pltpu.SemaphoreType.DMA,
        ],
    )
    def k(tab_hbm, idx_hbm, out_hbm, idx_v, rows_a, rows_b, sem_a, sem_b):
        w = _wid()
        pltpu.sync_copy(idx_hbm.at[w], idx_v)
        base = w * (nchunks * chunk)

        pltpu.async_copy(tab_hbm.at[idx_v.at[0]], rows_a, sem_a)

        @pl.loop(0, nchunks, step=2)
        def _(j):
            pltpu.make_async_copy(tab_hbm.at[idx_v.at[j]], rows_a, sem_a).wait()
            pltpu.async_copy(tab_hbm.at[idx_v.at[j + 1]], rows_b, sem_b)
            pltpu.sync_copy(rows_a, out_hbm.at[pl.ds(base + j * chunk, chunk)])
            pltpu.make_async_copy(tab_hbm.at[idx_v.at[j + 1]], rows_b,
                                  sem_b).wait()

            @pl.when(j + 2 < nchunks)
            def _():
                pltpu.async_copy(tab_hbm.at[idx_v.at[j + 2]], rows_a, sem_a)

            pltpu.sync_copy(rows_b,
                            out_hbm.at[pl.ds(base + (j + 1) * chunk, chunk)])

    return k(table, idx_groups)


def _sc_degree(dst_groups, zeros, ones):
    @functools.partial(
        pl.kernel,
        mesh=_mesh(),
        out_type=jax.ShapeDtypeStruct((NC, NPAD, 128), jnp.float32),
        scratch_types=[
            pltpu.VMEM((ED_CHUNKS, CHUNK), jnp.int32),
            pltpu.VMEM((CHUNK, 128), jnp.float32),
            pltpu.VMEM_SHARED((NPAD, 128), jnp.float32),
        ],
    )
    def k(dst_hbm, z_hbm, ones_hbm, out_hbm, idx_v, ones_v, acc):
        c = lax.axis_index("c")
        s = lax.axis_index("s")
        w = s * NC + c
        pltpu.sync_copy(dst_hbm.at[w], idx_v)
        pltpu.sync_copy(ones_hbm, ones_v)

        @pl.when(s == 0)
        def _():
            pltpu.sync_copy(z_hbm, acc)

        plsc.subcore_barrier()

        @pl.loop(0, ED_CHUNKS)
        def _(j):
            pltpu.sync_copy(ones_v, acc.at[idx_v.at[j]], add=True)

        plsc.subcore_barrier()
        rows = NPAD // NS
        pltpu.sync_copy(acc.at[pl.ds(s * rows, rows)],
                        out_hbm.at[c, pl.ds(s * rows, rows)])

    return k(dst_groups, zeros, ones)


def _sc_edge_agg(table, src_groups, dst_groups, zeros):
    d = table.shape[1]

    @functools.partial(
        pl.kernel,
        mesh=_mesh(),
        out_type=jax.ShapeDtypeStruct((NC, NPAD, d), jnp.float32),
        scratch_types=[
            pltpu.VMEM((ED_CHUNKS, CHUNK), jnp.int32),
            pltpu.VMEM((ED_CHUNKS, CHUNK), jnp.int32),
            pltpu.VMEM((CHUNK, d), jnp.float32),
            pltpu.VMEM((CHUNK, d), jnp.float32),
            pltpu.VMEM_SHARED((NPAD, d), jnp.float32),
            pltpu.SemaphoreType.DMA,
            pltpu.SemaphoreType.DMA,
        ],
    )
    def k(tab_hbm, src_hbm, dst_hbm, z_hbm, out_hbm,
          src_v, dst_v, rows_a, rows_b, acc, sem_a, sem_b):
        c = lax.axis_index("c")
        s = lax.axis_index("s")
        w = s * NC + c
        pltpu.sync_copy(src_hbm.at[w], src_v)
        pltpu.sync_copy(dst_hbm.at[w], dst_v)

        @pl.when(s == 0)
        def _():
            pltpu.sync_copy(z_hbm, acc)

        plsc.subcore_barrier()

        pltpu.async_copy(tab_hbm.at[src_v.at[0]], rows_a, sem_a)

        @pl.loop(0, ED_CHUNKS, step=2)
        def _(j):
            pltpu.make_async_copy(tab_hbm.at[src_v.at[j]], rows_a, sem_a).wait()
            pltpu.async_copy(tab_hbm.at[src_v.at[j + 1]], rows_b, sem_b)
            pltpu.sync_copy(rows_a, acc.at[dst_v.at[j]], add=True)
            pltpu.make_async_copy(tab_hbm.at[src_v.at[j + 1]], rows_b,
                                  sem_b).wait()

            @pl.when(j + 2 < ED_CHUNKS)
            def _():
                pltpu.async_copy(tab_hbm.at[src_v.at[j + 2]], rows_a, sem_a)

            pltpu.sync_copy(rows_b, acc.at[dst_v.at[j + 1]], add=True)

        plsc.subcore_barrier()
        rows = NPAD // NS
        pltpu.sync_copy(acc.at[pl.ds(s * rows, rows)],
                        out_hbm.at[c, pl.ds(s * rows, rows)])

    return k(table, src_groups, dst_groups, zeros)


def _tc_user_encoder(uf, w1t, b1, w2t, b2):
    def body(uf_ref, w1_ref, b1_ref, w2_ref, b2_ref, o_ref):
        uh = jnp.maximum(
            jnp.dot(uf_ref[...], w1_ref[...],
                    preferred_element_type=jnp.float32) + b1_ref[...], 0.0)
        o_ref[...] = jnp.dot(uh, w2_ref[...],
                             preferred_element_type=jnp.float32) + b2_ref[...]

    return pl.pallas_call(
        body,
        out_shape=jax.ShapeDtypeStruct((N_USERS, 100), jnp.float32),
    )(uf, w1t, b1, w2t, b2)


def _tc_gru(emb, h0p, wih0t, whh0t, bih0, bhh0, wih1t, whh1t, bih1, bhh1):
    g3 = 3 * TWEET_OUT

    def gates(gi, gh, h):
        r = jax.nn.sigmoid(gi[:, 0:100] + gh[:, 0:100])
        z = jax.nn.sigmoid(gi[:, 100:200] + gh[:, 100:200])
        ng = jnp.tanh(gi[:, 200:300] + r * gh[:, 200:300])
        return (1.0 - z) * ng + z * h

    def body(emb_ref, h0_ref, wih0_ref, whh0_ref, bih0_ref, bhh0_ref,
             wih1_ref, whh1_ref, bih1_ref, bhh1_ref, o_ref, gi_s, out0_s):
        x = emb_ref[...].reshape(SEQ_LEN * TBLK, EMBED_DIM)
        gi0 = jnp.dot(x, wih0_ref[...],
                      preferred_element_type=jnp.float32) + bih0_ref[...]
        gi_s[...] = gi0.reshape(SEQ_LEN, TBLK, g3)

        def step0(t, h):
            gh = jnp.dot(h, whh0_ref[...],
                         preferred_element_type=jnp.float32) + bhh0_ref[...]
            hn = gates(gi_s[t], gh, h)
            out0_s[t] = hn
            return hn

        lax.fori_loop(0, SEQ_LEN, step0, h0_ref[0])

        o0 = out0_s[...].reshape(SEQ_LEN * TBLK, TWEET_OUT)
        gi1 = jnp.dot(o0, wih1_ref[...],
                      preferred_element_type=jnp.float32) + bih1_ref[...]
        gi_s[...] = gi1.reshape(SEQ_LEN, TBLK, g3)

        def step1(t, h):
            gh = jnp.dot(h, whh1_ref[...],
                         preferred_element_type=jnp.float32) + bhh1_ref[...]
            return gates(gi_s[t], gh, h)

        o_ref[...] = lax.fori_loop(0, SEQ_LEN, step1, h0_ref[1])

    grid = TPAD // TBLK
    return pl.pallas_call(
        body,
        grid=(grid,),
        in_specs=[
            pl.BlockSpec((SEQ_LEN, TBLK, EMBED_DIM), lambda i: (0, i, 0)),
            pl.BlockSpec((2, TBLK, TWEET_OUT), lambda i: (0, i, 0)),
            pl.BlockSpec((EMBED_DIM, g3), lambda i: (0, 0)),
            pl.BlockSpec((TWEET_OUT, g3), lambda i: (0, 0)),
            pl.BlockSpec((1, g3), lambda i: (0, 0)),
            pl.BlockSpec((1, g3), lambda i: (0, 0)),
            pl.BlockSpec((TWEET_OUT, g3), lambda i: (0, 0)),
            pl.BlockSpec((TWEET_OUT, g3), lambda i: (0, 0)),
            pl.BlockSpec((1, g3), lambda i: (0, 0)),
            pl.BlockSpec((1, g3), lambda i: (0, 0)),
        ],
        out_specs=pl.BlockSpec((TBLK, TWEET_OUT), lambda i: (i, 0)),
        out_shape=jax.ShapeDtypeStruct((TPAD, TWEET_OUT), jnp.float32),
        scratch_shapes=[
            pltpu.VMEM((SEQ_LEN, TBLK, g3), jnp.float32),
            pltpu.VMEM((SEQ_LEN, TBLK, TWEET_OUT), jnp.float32),
        ],
    )(emb, h0p, wih0t, whh0t, bih0, bhh0, wih1t, whh1t, bih1, bhh1)


def _tc_stage_a(x_in, w1t, degp):
    def body(x_ref, w_ref, deg_ref, xw_ref, dinv_ref):
        deg = deg_ref[0, :, 0:1] + deg_ref[1, :, 0:1] + 1.0
        dinv = lax.rsqrt(deg)
        dinv_ref[...] = dinv
        xw = jnp.dot(x_ref[...], w_ref[...], preferred_element_type=jnp.float32)
        xw_ref[...] = dinv * xw

    return pl.pallas_call(
        body,
        out_shape=(
            jax.ShapeDtypeStruct((NPAD, 128), jnp.float32),
            jax.ShapeDtypeStruct((NPAD, 1), jnp.float32),
        ),
    )(x_in, w1t, degp)


def _tc_stage_b(agg1, xw1s, dinv, b1, w2mt, w2lt):
    def body(agg_ref, xw_ref, dinv_ref, b1_ref, wm_ref, wl_ref, om_ref, ol_ref):
        dinv = dinv_ref[...]
        pre = dinv * (agg_ref[0] + agg_ref[1] + xw_ref[...]) + b1_ref[...]
        x = jnp.where(pre > 0.0, pre, jnp.exp(jnp.minimum(pre, 0.0)) - 1.0)
        x = x[:, 0:H1]
        om_ref[...] = dinv * jnp.dot(x, wm_ref[...],
                                     preferred_element_type=jnp.float32)
        ol_ref[...] = dinv * jnp.dot(x, wl_ref[...],
                                     preferred_element_type=jnp.float32)

    return pl.pallas_call(
        body,
        out_shape=(
            jax.ShapeDtypeStruct((NPAD, 128), jnp.float32),
            jax.ShapeDtypeStruct((NPAD, 128), jnp.float32),
        ),
    )(agg1, xw1s, dinv, b1, w2mt, w2lt)


def _tc_stage_c(agg_m, agg_l, xwm, xwl, dinv, bm, bl, epsp):
    def body(aggm_ref, aggl_ref, xwm_ref, xwl_ref, dinv_ref, bm_ref, bl_ref,
             eps_ref, z_ref, zbf_ref, kl_ref):
        dinv = dinv_ref[...]
        pre_m = dinv * (aggm_ref[0] + aggm_ref[1] + xwm_ref[...]) + bm_ref[...]
        pre_l = dinv * (aggl_ref[0] + aggl_ref[1] + xwl_ref[...]) + bl_ref[...]
        mu = jnp.maximum(pre_m[:, 0:100], 0.0)
        logvar = jnp.maximum(pre_l[:, 0:100], 0.0)
        rows = lax.broadcasted_iota(jnp.int32, (NPAD, 1), 0)
        rmask = rows < N_NODES
        z = mu + jnp.exp(logvar * 0.5) * eps_ref[...]
        z = jnp.where(rmask, z, 0.0)
        z_ref[...] = z
        zbf_ref[...] = jnp.concatenate(
            [z, jnp.zeros((NPAD, 28), jnp.float32)], axis=1)
        klds = -0.5 * (1.0 + logvar - mu * mu - jnp.exp(logvar))
        klds = jnp.where(rmask, klds, 0.0)
        kl_ref[0, 0] = jnp.sum(klds)

    return pl.pallas_call(
        body,
        out_shape=(
            jax.ShapeDtypeStruct((NPAD, 100), jnp.float32),
            jax.ShapeDtypeStruct((NPAD, 128), jnp.float32),
            jax.ShapeDtypeStruct((1, 1), jnp.float32),
        ),
        out_specs=(
            pl.BlockSpec(memory_space=pltpu.VMEM),
            pl.BlockSpec(memory_space=pltpu.VMEM),
            pl.BlockSpec(memory_space=pltpu.SMEM),
        ),
    )(agg_m, agg_l, xwm, xwl, dinv, bm, bl, epsp)


def _tc_decoder(zbf):
    g = NPAD // NBLK

    def body(zi_ref, zj_ref, s1_ref, s2_ref):
        i = pl.program_id(0)
        j = pl.program_id(1)

        @pl.when(jnp.logical_and(i == 0, j == 0))
        def _():
            s1_ref[0, 0] = 0.0
            s2_ref[0, 0] = 0.0

        @pl.when(j >= i)
        def _():
            p = lax.dot_general(zi_ref[...].astype(jnp.bfloat16),
                                zj_ref[...].astype(jnp.bfloat16),
                                (((1,), (1,)), ((), ())),
                                preferred_element_type=jnp.float32)
            rows = i * NBLK + lax.broadcasted_iota(jnp.int32, (NBLK, NBLK), 0)
            cols = j * NBLK + lax.broadcasted_iota(jnp.int32, (NBLK, NBLK), 1)
            valid = jnp.logical_and(rows < N_NODES, cols < N_NODES)
            wt = jnp.where(i == j, 1.0, 2.0)
            sig = jax.nn.sigmoid(p)
            s1_ref[0, 0] += wt * jnp.sum(jnp.where(valid, sig, 0.0))
            pr = jnp.clip(sig, 1e-7, 1.0 - 1e-7)
            t2 = -jnp.log(1.0 - pr)
            s2_ref[0, 0] += wt * jnp.sum(jnp.where(valid, t2, 0.0))

    return pl.pallas_call(
        body,
        grid=(g, g),
        in_specs=[
            pl.BlockSpec((NBLK, 128), lambda i, j: (i, 0)),
            pl.BlockSpec((NBLK, 128), lambda i, j: (j, 0)),
        ],
        out_specs=(
            pl.BlockSpec(memory_space=pltpu.SMEM),
            pl.BlockSpec(memory_space=pltpu.SMEM),
        ),
        out_shape=(
            jax.ShapeDtypeStruct((1, 1), jnp.float32),
            jax.ShapeDtypeStruct((1, 1), jnp.float32),
        ),
    )(zbf, zbf)


def _tc_corrections(za, zb, keys3):
    eblk = 512
    g = E_ENT // eblk

    def body(za_ref, zb_ref, k_ref, o_ref):
        i = pl.program_id(0)

        @pl.when(i == 0)
        def _():
            o_ref[0, 0] = 0.0

        prod = za_ref[...] * zb_ref[...]
        d = lax.dot_general(prod, jnp.ones((128, 1), jnp.float32),
                            (((1,), (0,)), ((), ())),
                            preferred_element_type=jnp.float32)
        k0 = k_ref[0]
        singleton = jnp.logical_and(k0 != k_ref[1], k0 != k_ref[2])
        pos = i * eblk + lax.broadcasted_iota(jnp.int32, (eblk, 1), 0)
        valid = pos < N_ENT
        p = jnp.clip(jax.nn.sigmoid(d), 1e-7, 1.0 - 1e-7)
        logp = jnp.log(p)
        log1mp = jnp.log(1.0 - p)
        b0 = -log1mp
        ell = log1mp - logp
        contrib = jnp.where(singleton,
                            POS_WEIGHT * (b0 + ell) - b0, ell)
        o_ref[0, 0] += jnp.sum(jnp.where(valid, contrib, 0.0))

    return pl.pallas_call(
        body,
        grid=(g,),
        in_specs=[
            pl.BlockSpec((eblk, 128), lambda i: (i, 0)),
            pl.BlockSpec((eblk, 128), lambda i: (i, 0)),
            pl.BlockSpec((3, eblk, 1), lambda i: (0, i, 0)),
        ],
        out_specs=pl.BlockSpec(memory_space=pltpu.SMEM),
        out_shape=jax.ShapeDtypeStruct((1, 1), jnp.float32),
    )(za, zb, keys3)


def kernel(user_feats, graph_node_features, graph_edge_index, indices,
           ue_W1, ue_b1, ue_W2, ue_b2, emb_table,
           W_ih0, W_hh0, b_ih0, b_hh0, W_ih1, W_hh1, b_ih1, b_hh1, h0,
           conv1_W, conv1_b, mean_W, mean_b, logvar_W, logvar_b, eps):
    bs = indices.shape[0]
    f32 = jnp.float32

    tok = graph_node_features.astype(jnp.int32).T
    tok = jnp.pad(tok, ((0, 0), (0, TPAD - N_TWEETS)))
    tok_g = tok.reshape(NW, EG_CHUNKS, EG_CHUNK)
    emb = _sc_gather(emb_table, tok_g, SEQ_LEN * TPAD, EG_CHUNKS, EG_CHUNK)
    emb = emb.reshape(SEQ_LEN, TPAD, EMBED_DIM)

    user_emb = _tc_user_encoder(
        user_feats, ue_W1.T, ue_b1.reshape(1, -1), ue_W2.T, ue_b2.reshape(1, -1))

    h0p = jnp.pad(h0, ((0, 0), (0, TPAD - N_TWEETS), (0, 0)))
    hn = _tc_gru(emb, h0p,
                 W_ih0.T, W_hh0.T, b_ih0.reshape(1, -1), b_hh0.reshape(1, -1),
                 W_ih1.T, W_hh1.T, b_ih1.reshape(1, -1), b_hh1.reshape(1, -1))

    src = graph_edge_index[0].astype(jnp.int32)
    dst = graph_edge_index[1].astype(jnp.int32)
    src_g = src.reshape(NW, ED_CHUNKS, CHUNK)
    dst_g = dst.reshape(NW, ED_CHUNKS, CHUNK)

    ones128 = jnp.ones((CHUNK, 128), f32)
    degp = _sc_degree(dst_g, jnp.zeros((NPAD, 128), f32), ones128)

    x_in = jnp.concatenate([hn[:bs], user_emb, hn[bs:N_TWEETS]], axis=0)
    x_in = jnp.pad(x_in, ((0, NPAD - N_NODES), (0, 0)))
    w1t = jnp.pad(conv1_W.T, ((0, 0), (0, 128 - H1)))
    xw1s, dinv = _tc_stage_a(x_in, w1t, degp)
    agg1 = _sc_edge_agg(xw1s, src_g, dst_g, jnp.zeros((NPAD, 128), f32))

    w2mt = jnp.pad(mean_W.T, ((0, 0), (0, 28)))
    w2lt = jnp.pad(logvar_W.T, ((0, 0), (0, 28)))
    b1p = jnp.pad(conv1_b.reshape(1, -1), ((0, 0), (0, 128 - H1)))
    xwm, xwl = _tc_stage_b(agg1, xw1s, dinv, b1p, w2mt, w2lt)
    zeros128 = jnp.zeros((NPAD, 128), f32)
    agg_m = _sc_edge_agg(xwm, src_g, dst_g, zeros128)
    agg_l = _sc_edge_agg(xwl, src_g, dst_g, zeros128)

    bm = jnp.pad(mean_b.reshape(1, -1), ((0, 0), (0, 28)))
    bl = jnp.pad(logvar_b.reshape(1, -1), ((0, 0), (0, 28)))
    epsp = jnp.pad(eps, ((0, NPAD - N_NODES), (0, 0)))
    zf, zbf, klsum = _tc_stage_c(agg_m, agg_l, xwm, xwl, dinv, bm, bl, epsp)
    kl_loss = (klsum / float(N_NODES)).reshape(1)

    diag = jnp.arange(N_NODES, dtype=jnp.int32) * (N_NODES + 1)
    keys = jnp.concatenate([
        src * N_NODES + dst, diag,
        jnp.full((E_ENT - N_ENT,), jnp.iinfo(jnp.int32).max, jnp.int32)])
    keys = jnp.sort(keys)
    kc = jnp.minimum(keys, N_NODES * N_NODES - 1)
    a_idx = (kc // N_NODES).reshape(NW, EN_CHUNKS, EN_CHUNK)
    b_idx = (kc % N_NODES).reshape(NW, EN_CHUNKS, EN_CHUNK)
    kprev = jnp.concatenate([jnp.full((1,), -1, jnp.int32), keys[:-1]])
    knext = jnp.concatenate([keys[1:], jnp.full((1,), -2, jnp.int32)])
    keys3 = jnp.stack([keys, kprev, knext]).reshape(3, E_ENT, 1)

    za = _sc_gather(zbf, a_idx, E_ENT, EN_CHUNKS, EN_CHUNK)
    zb = _sc_gather(zbf, b_idx, E_ENT, EN_CHUNKS, EN_CHUNK)

    s1, s2 = _tc_decoder(zbf)
    corr = _tc_corrections(za, zb, keys3)

    s1v = s1[0, 0]
    norm = N2 / ((N2 - s1v) * 2.0)
    rec_loss = norm * (s2[0, 0] + corr[0, 0]) / N2

    return (zf[:bs, :], kl_loss, rec_loss)


# speedup vs baseline: 4.5142x; 1.0001x over previous
import functools

import jax
import jax.numpy as jnp
from jax import lax
from jax.experimental import pallas as pl
from jax.experimental.pallas import tpu as pltpu
from jax.experimental.pallas import tpu_sc as plsc

N_TWEETS = 3000
N_USERS = 3000
N_NODES = 6000
N_EDGES = 192000
SEQ_LEN = 30
EMBED_DIM = 128
TWEET_OUT = 100
H1 = 64
H2 = 100
BS = 1024

NPAD = 6144
TPAD = 3072
TBLK = 512
NBLK = 512
N2 = float(N_NODES) * float(N_NODES)

NC, NS = 2, 16
NW = NC * NS
CHUNK = 120

EG_CHUNKS = 24
EG_CHUNK = 120
ED_CHUNKS = 50
E_ENT = 199680
EN_CHUNKS = 52
EN_CHUNK = 120
N_ENT = N_EDGES + N_NODES
POS_WEIGHT = (N2 - N_ENT) / N_ENT

@functools.cache
def _mesh():
    return plsc.VectorSubcoreMesh(core_axis_name="c", subcore_axis_name="s",
                                  num_cores=NC)


def _wid():
    return lax.axis_index("s") * NC + lax.axis_index("c")


def _sc_gather(table, idx_groups, out_rows, nchunks, chunk):
    d = table.shape[1]

    assert nchunks % 2 == 0

    @functools.partial(
        pl.kernel,
        mesh=_mesh(),
        out_type=jax.ShapeDtypeStruct((out_rows, d), table.dtype),
        scratch_types=[
            pltpu.VMEM((nchunks, chunk), jnp.int32),
            pltpu.VMEM((chunk, d), table.dtype),
            pltpu.VMEM((chunk, d), table.dtype),
            pltpu.SemaphoreType.DMA,
            pltpu.SemaphoreType.DMA,
        ],
    )
    def k(tab_hbm, idx_hbm, out_hbm, idx_v, rows_a, rows_b, sem_a, sem_b):
        w = _wid()
        pltpu.sync_copy(idx_hbm.at[w], idx_v)
        base = w * (nchunks * chunk)

        pltpu.async_copy(tab_hbm.at[idx_v.at[0]], rows_a, sem_a)

        @pl.loop(0, nchunks, step=2)
        def _(j):
            pltpu.make_async_copy(tab_hbm.at[idx_v.at[j]], rows_a, sem_a).wait()
            pltpu.async_copy(tab_hbm.at[idx_v.at[j + 1]], rows_b, sem_b)
            pltpu.sync_copy(rows_a, out_hbm.at[pl.ds(base + j * chunk, chunk)])
            pltpu.make_async_copy(tab_hbm.at[idx_v.at[j + 1]], rows_b,
                                  sem_b).wait()

            @pl.when(j + 2 < nchunks)
            def _():
                pltpu.async_copy(tab_hbm.at[idx_v.at[j + 2]], rows_a, sem_a)

            pltpu.sync_copy(rows_b,
                            out_hbm.at[pl.ds(base + (j + 1) * chunk, chunk)])

    return k(table, idx_groups)


def _sc_degree(dst_groups, zeros, ones):
    @functools.partial(
        pl.kernel,
        mesh=_mesh(),
        out_type=jax.ShapeDtypeStruct((NC, NPAD, 128), jnp.float32),
        scratch_types=[
            pltpu.VMEM((ED_CHUNKS, CHUNK), jnp.int32),
            pltpu.VMEM((CHUNK, 128), jnp.float32),
            pltpu.VMEM_SHARED((NPAD, 128), jnp.float32),
        ],
    )
    def k(dst_hbm, z_hbm, ones_hbm, out_hbm, idx_v, ones_v, acc):
        c = lax.axis_index("c")
        s = lax.axis_index("s")
        w = s * NC + c
        pltpu.sync_copy(dst_hbm.at[w], idx_v)
        pltpu.sync_copy(ones_hbm, ones_v)

        @pl.when(s == 0)
        def _():
            pltpu.sync_copy(z_hbm, acc)

        plsc.subcore_barrier()

        @pl.loop(0, ED_CHUNKS)
        def _(j):
            pltpu.sync_copy(ones_v, acc.at[idx_v.at[j]], add=True)

        plsc.subcore_barrier()
        rows = NPAD // NS
        pltpu.sync_copy(acc.at[pl.ds(s * rows, rows)],
                        out_hbm.at[c, pl.ds(s * rows, rows)])

    return k(dst_groups, zeros, ones)


def _sc_edge_agg(table, src_groups, dst_groups, zeros):
    d = table.shape[1]

    @functools.partial(
        pl.kernel,
        mesh=_mesh(),
        out_type=jax.ShapeDtypeStruct((NC, NPAD, d), jnp.float32),
        scratch_types=[
            pltpu.VMEM((ED_CHUNKS, CHUNK), jnp.int32),
            pltpu.VMEM((ED_CHUNKS, CHUNK), jnp.int32),
            pltpu.VMEM((CHUNK, d), jnp.float32),
            pltpu.VMEM((CHUNK, d), jnp.float32),
            pltpu.VMEM_SHARED((NPAD, d), jnp.float32),
            pltpu.SemaphoreType.DMA,
            pltpu.SemaphoreType.DMA,
        ],
    )
    def k(tab_hbm, src_hbm, dst_hbm, z_hbm, out_hbm,
          src_v, dst_v, rows_a, rows_b, acc, sem_a, sem_b):
        c = lax.axis_index("c")
        s = lax.axis_index("s")
        w = s * NC + c
        pltpu.sync_copy(src_hbm.at[w], src_v)
        pltpu.sync_copy(dst_hbm.at[w], dst_v)

        @pl.when(s == 0)
        def _():
            pltpu.sync_copy(z_hbm, acc)

        plsc.subcore_barrier()

        pltpu.async_copy(tab_hbm.at[src_v.at[0]], rows_a, sem_a)

        @pl.loop(0, ED_CHUNKS, step=2)
        def _(j):
            pltpu.make_async_copy(tab_hbm.at[src_v.at[j]], rows_a, sem_a).wait()
            pltpu.async_copy(tab_hbm.at[src_v.at[j + 1]], rows_b, sem_b)
            pltpu.sync_copy(rows_a, acc.at[dst_v.at[j]], add=True)
            pltpu.make_async_copy(tab_hbm.at[src_v.at[j + 1]], rows_b,
                                  sem_b).wait()

            @pl.when(j + 2 < ED_CHUNKS)
            def _():
                pltpu.async_copy(tab_hbm.at[src_v.at[j + 2]], rows_a, sem_a)

            pltpu.sync_copy(rows_b, acc.at[dst_v.at[j + 1]], add=True)

        plsc.subcore_barrier()
        rows = NPAD // NS
        pltpu.sync_copy(acc.at[pl.ds(s * rows, rows)],
                        out_hbm.at[c, pl.ds(s * rows, rows)])

    return k(table, src_groups, dst_groups, zeros)


def _tc_user_encoder(uf, w1t, b1, w2t, b2):
    def body(uf_ref, w1_ref, b1_ref, w2_ref, b2_ref, o_ref):
        uh = jnp.maximum(
            jnp.dot(uf_ref[...], w1_ref[...],
                    preferred_element_type=jnp.float32) + b1_ref[...], 0.0)
        o_ref[...] = jnp.dot(uh, w2_ref[...],
                             preferred_element_type=jnp.float32) + b2_ref[...]

    return pl.pallas_call(
        body,
        out_shape=jax.ShapeDtypeStruct((N_USERS, 100), jnp.float32),
    )(uf, w1t, b1, w2t, b2)


def _tc_gru(emb, h0p, wih0t, whh0t, bih0, bhh0, wih1t, whh1t, bih1, bhh1):
    g3 = 3 * TWEET_OUT

    def gates(gi, gh, h):
        r = jax.nn.sigmoid(gi[:, 0:100] + gh[:, 0:100])
        z = jax.nn.sigmoid(gi[:, 100:200] + gh[:, 100:200])
        ng = jnp.tanh(gi[:, 200:300] + r * gh[:, 200:300])
        return (1.0 - z) * ng + z * h

    def body(emb_ref, h0_ref, wih0_ref, whh0_ref, bih0_ref, bhh0_ref,
             wih1_ref, whh1_ref, bih1_ref, bhh1_ref, o_ref, gi_s, out0_s):
        x = emb_ref[...].reshape(SEQ_LEN * TBLK, EMBED_DIM)
        gi0 = jnp.dot(x.astype(jnp.bfloat16),
                      wih0_ref[...].astype(jnp.bfloat16),
                      preferred_element_type=jnp.float32) + bih0_ref[...]
        gi_s[...] = gi0.reshape(SEQ_LEN, TBLK, g3)

        whh0b = whh0_ref[...].astype(jnp.bfloat16)

        def step0(t, h):
            gh = jnp.dot(h.astype(jnp.bfloat16), whh0b,
                         preferred_element_type=jnp.float32) + bhh0_ref[...]
            hn = gates(gi_s[t], gh, h)
            out0_s[t] = hn
            return hn

        lax.fori_loop(0, SEQ_LEN, step0, h0_ref[0])

        o0 = out0_s[...].reshape(SEQ_LEN * TBLK, TWEET_OUT)
        gi1 = jnp.dot(o0.astype(jnp.bfloat16),
                      wih1_ref[...].astype(jnp.bfloat16),
                      preferred_element_type=jnp.float32) + bih1_ref[...]
        gi_s[...] = gi1.reshape(SEQ_LEN, TBLK, g3)

        whh1b = whh1_ref[...].astype(jnp.bfloat16)

        def step1(t, h):
            gh = jnp.dot(h.astype(jnp.bfloat16), whh1b,
                         preferred_element_type=jnp.float32) + bhh1_ref[...]
            return gates(gi_s[t], gh, h)

        o_ref[...] = lax.fori_loop(0, SEQ_LEN, step1, h0_ref[1])

    grid = TPAD // TBLK
    return pl.pallas_call(
        body,
        grid=(grid,),
        in_specs=[
            pl.BlockSpec((SEQ_LEN, TBLK, EMBED_DIM), lambda i: (0, i, 0)),
            pl.BlockSpec((2, TBLK, TWEET_OUT), lambda i: (0, i, 0)),
            pl.BlockSpec((EMBED_DIM, g3), lambda i: (0, 0)),
            pl.BlockSpec((TWEET_OUT, g3), lambda i: (0, 0)),
            pl.BlockSpec((1, g3), lambda i: (0, 0)),
            pl.BlockSpec((1, g3), lambda i: (0, 0)),
            pl.BlockSpec((TWEET_OUT, g3), lambda i: (0, 0)),
            pl.BlockSpec((TWEET_OUT, g3), lambda i: (0, 0)),
            pl.BlockSpec((1, g3), lambda i: (0, 0)),
            pl.BlockSpec((1, g3), lambda i: (0, 0)),
        ],
        out_specs=pl.BlockSpec((TBLK, TWEET_OUT), lambda i: (i, 0)),
        out_shape=jax.ShapeDtypeStruct((TPAD, TWEET_OUT), jnp.float32),
        scratch_shapes=[
            pltpu.VMEM((SEQ_LEN, TBLK, g3), jnp.float32),
            pltpu.VMEM((SEQ_LEN, TBLK, TWEET_OUT), jnp.float32),
        ],
    )(emb, h0p, wih0t, whh0t, bih0, bhh0, wih1t, whh1t, bih1, bhh1)


def _tc_stage_a(x_in, w1t, degp):
    def body(x_ref, w_ref, deg_ref, xw_ref, dinv_ref):
        deg = deg_ref[0, :, 0:1] + deg_ref[1, :, 0:1] + 1.0
        dinv = lax.rsqrt(deg)
        dinv_ref[...] = dinv
        xw = jnp.dot(x_ref[...], w_ref[...], preferred_element_type=jnp.float32)
        xw_ref[...] = dinv * xw

    return pl.pallas_call(
        body,
        out_shape=(
            jax.ShapeDtypeStruct((NPAD, 128), jnp.float32),
            jax.ShapeDtypeStruct((NPAD, 1), jnp.float32),
        ),
    )(x_in, w1t, degp)


def _tc_stage_b(agg1, xw1s, dinv, b1, w2mt, w2lt):
    def body(agg_ref, xw_ref, dinv_ref, b1_ref, wm_ref, wl_ref, om_ref, ol_ref):
        dinv = dinv_ref[...]
        pre = dinv * (agg_ref[0] + agg_ref[1] + xw_ref[...]) + b1_ref[...]
        x = jnp.where(pre > 0.0, pre, jnp.exp(jnp.minimum(pre, 0.0)) - 1.0)
        x = x[:, 0:H1]
        om_ref[...] = dinv * jnp.dot(x, wm_ref[...],
                                     preferred_element_type=jnp.float32)
        ol_ref[...] = dinv * jnp.dot(x, wl_ref[...],
                                     preferred_element_type=jnp.float32)

    return pl.pallas_call(
        body,
        out_shape=(
            jax.ShapeDtypeStruct((NPAD, 128), jnp.float32),
            jax.ShapeDtypeStruct((NPAD, 128), jnp.float32),
        ),
    )(agg1, xw1s, dinv, b1, w2mt, w2lt)


def _tc_stage_c(agg_m, agg_l, xwm, xwl, dinv, bm, bl, epsp):
    def body(aggm_ref, aggl_ref, xwm_ref, xwl_ref, dinv_ref, bm_ref, bl_ref,
             eps_ref, z_ref, zbf_ref, kl_ref):
        dinv = dinv_ref[...]
        pre_m = dinv * (aggm_ref[0] + aggm_ref[1] + xwm_ref[...]) + bm_ref[...]
        pre_l = dinv * (aggl_ref[0] + aggl_ref[1] + xwl_ref[...]) + bl_ref[...]
        mu = jnp.maximum(pre_m[:, 0:100], 0.0)
        logvar = jnp.maximum(pre_l[:, 0:100], 0.0)
        rows = lax.broadcasted_iota(jnp.int32, (NPAD, 1), 0)
        rmask = rows < N_NODES
        z = mu + jnp.exp(logvar * 0.5) * eps_ref[...]
        z = jnp.where(rmask, z, 0.0)
        z_ref[...] = z
        zbf_ref[...] = jnp.concatenate(
            [z, jnp.zeros((NPAD, 28), jnp.float32)], axis=1)
        klds = -0.5 * (1.0 + logvar - mu * mu - jnp.exp(logvar))
        klds = jnp.where(rmask, klds, 0.0)
        kl_ref[0, 0] = jnp.sum(klds)

    return pl.pallas_call(
        body,
        out_shape=(
            jax.ShapeDtypeStruct((NPAD, 100), jnp.float32),
            jax.ShapeDtypeStruct((NPAD, 128), jnp.float32),
            jax.ShapeDtypeStruct((1, 1), jnp.float32),
        ),
        out_specs=(
            pl.BlockSpec(memory_space=pltpu.VMEM),
            pl.BlockSpec(memory_space=pltpu.VMEM),
            pl.BlockSpec(memory_space=pltpu.SMEM),
        ),
    )(agg_m, agg_l, xwm, xwl, dinv, bm, bl, epsp)


def _tc_decoder(zbf):
    g = NPAD // NBLK

    def body(zi_ref, zj_ref, s1_ref, s2_ref):
        i = pl.program_id(0)
        j = pl.program_id(1)

        @pl.when(jnp.logical_and(i == 0, j == 0))
        def _():
            s1_ref[0, 0] = 0.0
            s2_ref[0, 0] = 0.0

        @pl.when(j >= i)
        def _():
            p = lax.dot_general(zi_ref[...].astype(jnp.bfloat16),
                                zj_ref[...].astype(jnp.bfloat16),
                                (((1,), (1,)), ((), ())),
                                preferred_element_type=jnp.float32)
            rows = i * NBLK + lax.broadcasted_iota(jnp.int32, (NBLK, NBLK), 0)
            cols = j * NBLK + lax.broadcasted_iota(jnp.int32, (NBLK, NBLK), 1)
            valid = jnp.logical_and(rows < N_NODES, cols < N_NODES)
            wt = jnp.where(i == j, 1.0, 2.0)
            sig = jax.nn.sigmoid(p)
            s1_ref[0, 0] += wt * jnp.sum(jnp.where(valid, sig, 0.0))
            pr = jnp.clip(sig, 1e-7, 1.0 - 1e-7)
            t2 = -jnp.log(1.0 - pr)
            s2_ref[0, 0] += wt * jnp.sum(jnp.where(valid, t2, 0.0))

    return pl.pallas_call(
        body,
        grid=(g, g),
        in_specs=[
            pl.BlockSpec((NBLK, 128), lambda i, j: (i, 0)),
            pl.BlockSpec((NBLK, 128), lambda i, j: (j, 0)),
        ],
        out_specs=(
            pl.BlockSpec(memory_space=pltpu.SMEM),
            pl.BlockSpec(memory_space=pltpu.SMEM),
        ),
        out_shape=(
            jax.ShapeDtypeStruct((1, 1), jnp.float32),
            jax.ShapeDtypeStruct((1, 1), jnp.float32),
        ),
    )(zbf, zbf)


def _tc_corrections(za, zb, keys3):
    eblk = 512
    g = E_ENT // eblk

    def body(za_ref, zb_ref, k_ref, o_ref):
        i = pl.program_id(0)

        @pl.when(i == 0)
        def _():
            o_ref[0, 0] = 0.0

        prod = za_ref[...] * zb_ref[...]
        d = lax.dot_general(prod, jnp.ones((128, 1), jnp.float32),
                            (((1,), (0,)), ((), ())),
                            preferred_element_type=jnp.float32)
        k0 = k_ref[0]
        singleton = jnp.logical_and(k0 != k_ref[1], k0 != k_ref[2])
        pos = i * eblk + lax.broadcasted_iota(jnp.int32, (eblk, 1), 0)
        valid = pos < N_ENT
        p = jnp.clip(jax.nn.sigmoid(d), 1e-7, 1.0 - 1e-7)
        logp = jnp.log(p)
        log1mp = jnp.log(1.0 - p)
        b0 = -log1mp
        ell = log1mp - logp
        contrib = jnp.where(singleton,
                            POS_WEIGHT * (b0 + ell) - b0, ell)
        o_ref[0, 0] += jnp.sum(jnp.where(valid, contrib, 0.0))

    return pl.pallas_call(
        body,
        grid=(g,),
        in_specs=[
            pl.BlockSpec((eblk, 128), lambda i: (i, 0)),
            pl.BlockSpec((eblk, 128), lambda i: (i, 0)),
            pl.BlockSpec((3, eblk, 1), lambda i: (0, i, 0)),
        ],
        out_specs=pl.BlockSpec(memory_space=pltpu.SMEM),
        out_shape=jax.ShapeDtypeStruct((1, 1), jnp.float32),
    )(za, zb, keys3)


def kernel(user_feats, graph_node_features, graph_edge_index, indices,
           ue_W1, ue_b1, ue_W2, ue_b2, emb_table,
           W_ih0, W_hh0, b_ih0, b_hh0, W_ih1, W_hh1, b_ih1, b_hh1, h0,
           conv1_W, conv1_b, mean_W, mean_b, logvar_W, logvar_b, eps):
    bs = indices.shape[0]
    f32 = jnp.float32

    tok = graph_node_features.astype(jnp.int32).T
    tok = jnp.pad(tok, ((0, 0), (0, TPAD - N_TWEETS)))
    tok_g = tok.reshape(NW, EG_CHUNKS, EG_CHUNK)
    emb = _sc_gather(emb_table, tok_g, SEQ_LEN * TPAD, EG_CHUNKS, EG_CHUNK)
    emb = emb.reshape(SEQ_LEN, TPAD, EMBED_DIM)

    user_emb = _tc_user_encoder(
        user_feats, ue_W1.T, ue_b1.reshape(1, -1), ue_W2.T, ue_b2.reshape(1, -1))

    h0p = jnp.pad(h0, ((0, 0), (0, TPAD - N_TWEETS), (0, 0)))
    hn = _tc_gru(emb, h0p,
                 W_ih0.T, W_hh0.T, b_ih0.reshape(1, -1), b_hh0.reshape(1, -1),
                 W_ih1.T, W_hh1.T, b_ih1.reshape(1, -1), b_hh1.reshape(1, -1))

    src = graph_edge_index[0].astype(jnp.int32)
    dst = graph_edge_index[1].astype(jnp.int32)
    src_g = src.reshape(NW, ED_CHUNKS, CHUNK)
    dst_g = dst.reshape(NW, ED_CHUNKS, CHUNK)

    ones128 = jnp.ones((CHUNK, 128), f32)
    degp = _sc_degree(dst_g, jnp.zeros((NPAD, 128), f32), ones128)

    x_in = jnp.concatenate([hn[:bs], user_emb, hn[bs:N_TWEETS]], axis=0)
    x_in = jnp.pad(x_in, ((0, NPAD - N_NODES), (0, 0)))
    w1t = jnp.pad(conv1_W.T, ((0, 0), (0, 128 - H1)))
    xw1s, dinv = _tc_stage_a(x_in, w1t, degp)
    agg1 = _sc_edge_agg(xw1s, src_g, dst_g, jnp.zeros((NPAD, 128), f32))

    w2mt = jnp.pad(mean_W.T, ((0, 0), (0, 28)))
    w2lt = jnp.pad(logvar_W.T, ((0, 0), (0, 28)))
    b1p = jnp.pad(conv1_b.reshape(1, -1), ((0, 0), (0, 128 - H1)))
    xwm, xwl = _tc_stage_b(agg1, xw1s, dinv, b1p, w2mt, w2lt)
    zeros128 = jnp.zeros((NPAD, 128), f32)
    agg_m = _sc_edge_agg(xwm, src_g, dst_g, zeros128)
    agg_l = _sc_edge_agg(xwl, src_g, dst_g, zeros128)

    bm = jnp.pad(mean_b.reshape(1, -1), ((0, 0), (0, 28)))
    bl = jnp.pad(logvar_b.reshape(1, -1), ((0, 0), (0, 28)))
    epsp = jnp.pad(eps, ((0, NPAD - N_NODES), (0, 0)))
    zf, zbf, klsum = _tc_stage_c(agg_m, agg_l, xwm, xwl, dinv, bm, bl, epsp)
    kl_loss = (klsum / float(N_NODES)).reshape(1)

    diag = jnp.arange(N_NODES, dtype=jnp.int32) * (N_NODES + 1)
    keys = jnp.concatenate([
        src * N_NODES + dst, diag,
        jnp.full((E_ENT - N_ENT,), jnp.iinfo(jnp.int32).max, jnp.int32)])
    keys = jnp.sort(keys)
    kc = jnp.minimum(keys, N_NODES * N_NODES - 1)
    a_idx = (kc // N_NODES).reshape(NW, EN_CHUNKS, EN_CHUNK)
    b_idx = (kc % N_NODES).reshape(NW, EN_CHUNKS, EN_CHUNK)
    kprev = jnp.concatenate([jnp.full((1,), -1, jnp.int32), keys[:-1]])
    knext = jnp.concatenate([keys[1:], jnp.full((1,), -2, jnp.int32)])
    keys3 = jnp.stack([keys, kprev, knext]).reshape(3, E_ENT, 1)

    za = _sc_gather(zbf, a_idx, E_ENT, EN_CHUNKS, EN_CHUNK)
    zb = _sc_gather(zbf, b_idx, E_ENT, EN_CHUNKS, EN_CHUNK)

    s1, s2 = _tc_decoder(zbf)
    corr = _tc_corrections(za, zb, keys3)

    s1v = s1[0, 0]
    norm = N2 / ((N2 - s1v) * 2.0)
    rec_loss = norm * (s2[0, 0] + corr[0, 0]) / N2

    return (zf[:bs, :], kl_loss, rec_loss)
